# baseline (speedup 1.0000x reference)
import jax
import jax.numpy as jnp
from jax.experimental import pallas as pl
from jax.experimental.pallas import tpu as pltpu

NS, T, E, H = 512, 64, 300, 256
TSUB = 2
NT = T // TSUB


def _sigmoid(x):
    return 0.5 * jnp.tanh(0.5 * x) + 0.5


def _gates_update(gates, c, Hh):
    i = _sigmoid(gates[:, :Hh])
    f = _sigmoid(gates[:, Hh:2 * Hh])
    g = jnp.tanh(gates[:, 2 * Hh:3 * Hh])
    o = _sigmoid(gates[:, 3 * Hh:])
    c_new = f * c + i * g
    h_new = o * jnp.tanh(c_new)
    return h_new, c_new


def _lstm_update(xt, h, c, Wih_ref, Whh_ref, b_ref):
    gates = (
        jnp.dot(xt, Wih_ref[...], preferred_element_type=jnp.float32)
        + jnp.dot(h, Whh_ref[...], preferred_element_type=jnp.float32)
        + b_ref[...]
    )
    return _gates_update(gates, c, Whh_ref.shape[0])


def _lstm_update_cat(xt, h, c, W_ref, b_ref, Hh):
    xh = jnp.concatenate([xt.astype(jnp.bfloat16), h.astype(jnp.bfloat16)],
                         axis=1)
    gates = jnp.dot(xh, W_ref[...], preferred_element_type=jnp.float32) + b_ref[...]
    return _gates_update(gates, c, Hh)


def _sent_kernel(lens_ref, xf_ref, xb_ref, Wf_ref, Uf_ref, bf_ref, Wb_ref,
                 Ub_ref, bb_ref, out_ref, hf, cf, hb, cb):
    tt = pl.program_id(0)

    @pl.when(tt == 0)
    def _init():
        hf[...] = jnp.zeros_like(hf)
        cf[...] = jnp.zeros_like(cf)
        hb[...] = jnp.zeros_like(hb)
        cb[...] = jnp.zeros_like(cb)

    lens = lens_ref[...]

    for k in range(TSUB):
        t = tt * TSUB + k
        tb = T - 1 - t

        h_new, c_new = _lstm_update(xf_ref[k], hf[...], cf[...],
                                    Wf_ref, Uf_ref, bf_ref)
        m = t < lens
        hf[...] = jnp.where(m, h_new, hf[...])
        cf[...] = jnp.where(m, c_new, cf[...])

        h_new, c_new = _lstm_update(xb_ref[TSUB - 1 - k], hb[...], cb[...],
                                    Wb_ref, Ub_ref, bb_ref)
        mb = tb < lens
        hb[...] = jnp.where(mb, h_new, hb[...])
        cb[...] = jnp.where(mb, c_new, cb[...])

    @pl.when(tt == NT - 1)
    def _emit():
        out_ref[:, :H] = hf[...]
        out_ref[:, H:] = hb[...]


def _doc_kernel(ridx_ref, offs_ref, maxc_ref, cnts_ref, semb_ref,
                Wf_ref, bf_ref, Wb_ref, bb_ref,
                fcW_ref, fcb_ref, out_ref, P, hf, cf, hb, cb):
    B = cnts_ref.shape[0]
    maxc = maxc_ref[0]
    cnts = cnts_ref[...]

    hf[...] = jnp.zeros_like(hf)
    cf[...] = jnp.zeros_like(cf)
    hb[...] = jnp.zeros_like(hb)
    cb[...] = jnp.zeros_like(cb)

    def stage(t, carry):
        for d in range(B):
            addr = jnp.minimum(offs_ref[d] + t, NS - 1)
            j = ridx_ref[addr]
            P[t, pl.ds(d, 1), :] = semb_ref[pl.ds(j, 1), :].astype(
                jnp.bfloat16)
        return carry

    jax.lax.fori_loop(0, maxc, stage, 0, unroll=False)

    def body(s, carry):
        xt = P[s]
        h_new, c_new = _lstm_update_cat(xt, hf[...], cf[...],
                                        Wf_ref, bf_ref, H)
        m = s < cnts
        hf[...] = jnp.where(m, h_new, hf[...])
        cf[...] = jnp.where(m, c_new, cf[...])
        tb = maxc - 1 - s
        xtb = P[tb]
        h_new, c_new = _lstm_update_cat(xtb, hb[...], cb[...],
                                        Wb_ref, bb_ref, H)
        mb = tb < cnts
        hb[...] = jnp.where(mb, h_new, hb[...])
        cb[...] = jnp.where(mb, c_new, cb[...])
        return carry

    jax.lax.fori_loop(0, maxc, body, 0, unroll=False)

    logits = (
        jnp.dot(hf[...], fcW_ref[:H, :], preferred_element_type=jnp.float32)
        + jnp.dot(hb[...], fcW_ref[H:, :], preferred_element_type=jnp.float32)
        + fcb_ref[0, 0]
    )
    out_ref[...] = _sigmoid(logits)


@jax.jit
def kernel(x, sWihf, sWhhf, sbf, sWihb, sWhhb, sbb, dWihf, dWhhf, dbf,
           dWihb, dWhhb, dbb, fcW, fcb, recover_idx, num_sent_per_document,
           sent_lengths):
    B = num_sent_per_document.shape[0]
    lens2d = sent_lengths.reshape(NS, 1)

    x_tm = jnp.transpose(x, (1, 0, 2))
    sent_emb_sorted = pl.pallas_call(
        _sent_kernel,
        grid=(NT,),
        in_specs=[
            pl.BlockSpec((NS, 1), lambda tt: (0, 0)),
            pl.BlockSpec((TSUB, NS, E), lambda tt: (tt, 0, 0)),
            pl.BlockSpec((TSUB, NS, E), lambda tt: (NT - 1 - tt, 0, 0)),
            pl.BlockSpec((E, 4 * H), lambda tt: (0, 0)),
            pl.BlockSpec((H, 4 * H), lambda tt: (0, 0)),
            pl.BlockSpec((1, 4 * H), lambda tt: (0, 0)),
            pl.BlockSpec((E, 4 * H), lambda tt: (0, 0)),
            pl.BlockSpec((H, 4 * H), lambda tt: (0, 0)),
            pl.BlockSpec((1, 4 * H), lambda tt: (0, 0)),
        ],
        out_specs=pl.BlockSpec((NS, 2 * H), lambda tt: (0, 0)),
        out_shape=jax.ShapeDtypeStruct((NS, 2 * H), jnp.float32),
        scratch_shapes=[pltpu.VMEM((NS, H), jnp.float32)] * 4,
    )(lens2d, x_tm, x_tm,
      sWihf, sWhhf, sbf.reshape(1, -1),
      sWihb, sWhhb, sbb.reshape(1, -1))

    counts = num_sent_per_document.astype(jnp.int32)
    offsets = jnp.concatenate(
        [jnp.zeros((1,), jnp.int32), jnp.cumsum(counts)[:-1]])
    maxc = jnp.max(counts).reshape(1)

    def catW(Wih, Whh):
        return jnp.concatenate([Wih, Whh]).astype(jnp.bfloat16)

    out2d = pl.pallas_call(
        _doc_kernel,
        in_specs=[
            pl.BlockSpec(memory_space=pltpu.SMEM),
            pl.BlockSpec(memory_space=pltpu.SMEM),
            pl.BlockSpec(memory_space=pltpu.SMEM),
            pl.BlockSpec((B, 1), lambda: (0, 0)),
            pl.BlockSpec((NS, 2 * H), lambda: (0, 0)),
            pl.BlockSpec((3 * H, 4 * H), lambda: (0, 0)),
            pl.BlockSpec((1, 4 * H), lambda: (0, 0)),
            pl.BlockSpec((3 * H, 4 * H), lambda: (0, 0)),
            pl.BlockSpec((1, 4 * H), lambda: (0, 0)),
            pl.BlockSpec((2 * H, 1), lambda: (0, 0)),
            pl.BlockSpec((1, 1), lambda: (0, 0)),
        ],
        out_specs=pl.BlockSpec((B, 1), lambda: (0, 0)),
        out_shape=jax.ShapeDtypeStruct((B, 1), jnp.float32),
        scratch_shapes=[pltpu.VMEM((NS, B, 2 * H), jnp.bfloat16)]
        + [pltpu.VMEM((B, H), jnp.float32)] * 4,
    )(recover_idx.astype(jnp.int32), offsets, maxc, counts.reshape(B, 1),
      sent_emb_sorted,
      catW(dWihf, dWhhf), dbf.reshape(1, -1),
      catW(dWihb, dWhhb), dbb.reshape(1, -1), fcW, fcb.reshape(1, 1))

    return out2d.reshape(-1)


# speedup vs baseline: 1.0194x; 1.0194x over previous
import jax
import jax.numpy as jnp
from jax.experimental import pallas as pl
from jax.experimental.pallas import tpu as pltpu

NS, T, E, H = 512, 64, 300, 256
TSUB = 4
NT = T // TSUB


def _sigmoid(x):
    return 0.5 * jnp.tanh(0.5 * x) + 0.5


def _gates_update(gates, c, Hh):
    i = _sigmoid(gates[:, :Hh])
    f = _sigmoid(gates[:, Hh:2 * Hh])
    g = jnp.tanh(gates[:, 2 * Hh:3 * Hh])
    o = _sigmoid(gates[:, 3 * Hh:])
    c_new = f * c + i * g
    h_new = o * jnp.tanh(c_new)
    return h_new, c_new


def _lstm_update(xt, h, c, Wih_ref, Whh_ref, b_ref):
    gates = (
        jnp.dot(xt, Wih_ref[...], preferred_element_type=jnp.float32)
        + jnp.dot(h, Whh_ref[...], preferred_element_type=jnp.float32)
        + b_ref[...]
    )
    return _gates_update(gates, c, Whh_ref.shape[0])


def _lstm_update_cat(xt, h, c, W_ref, b_ref, Hh):
    xh = jnp.concatenate([xt.astype(jnp.bfloat16), h.astype(jnp.bfloat16)],
                         axis=1)
    gates = jnp.dot(xh, W_ref[...], preferred_element_type=jnp.float32) + b_ref[...]
    return _gates_update(gates, c, Hh)


def _sent_kernel(lens_ref, xf_ref, xb_ref, Wf_ref, Uf_ref, bf_ref, Wb_ref,
                 Ub_ref, bb_ref, out_ref, hf, cf, hb, cb):
    tt = pl.program_id(0)

    @pl.when(tt == 0)
    def _init():
        hf[...] = jnp.zeros_like(hf)
        cf[...] = jnp.zeros_like(cf)
        hb[...] = jnp.zeros_like(hb)
        cb[...] = jnp.zeros_like(cb)

    lens = lens_ref[...]

    for k in range(TSUB):
        t = tt * TSUB + k
        tb = T - 1 - t

        h_new, c_new = _lstm_update(xf_ref[k], hf[...], cf[...],
                                    Wf_ref, Uf_ref, bf_ref)
        m = t < lens
        hf[...] = jnp.where(m, h_new, hf[...])
        cf[...] = jnp.where(m, c_new, cf[...])

        h_new, c_new = _lstm_update(xb_ref[TSUB - 1 - k], hb[...], cb[...],
                                    Wb_ref, Ub_ref, bb_ref)
        mb = tb < lens
        hb[...] = jnp.where(mb, h_new, hb[...])
        cb[...] = jnp.where(mb, c_new, cb[...])

    @pl.when(tt == NT - 1)
    def _emit():
        out_ref[:, :H] = hf[...]
        out_ref[:, H:] = hb[...]


def _doc_kernel(ridx_ref, offs_ref, maxc_ref, cnts_ref, semb_ref,
                Wf_ref, bf_ref, Wb_ref, bb_ref,
                fcW_ref, fcb_ref, out_ref, P, hf, cf, hb, cb):
    B = cnts_ref.shape[0]
    maxc = maxc_ref[0]
    cnts = cnts_ref[...]

    hf[...] = jnp.zeros_like(hf)
    cf[...] = jnp.zeros_like(cf)
    hb[...] = jnp.zeros_like(hb)
    cb[...] = jnp.zeros_like(cb)

    def stage(t, carry):
        for d in range(B):
            addr = jnp.minimum(offs_ref[d] + t, NS - 1)
            j = ridx_ref[addr]
            P[t, pl.ds(d, 1), :] = semb_ref[pl.ds(j, 1), :].astype(
                jnp.bfloat16)
        return carry

    jax.lax.fori_loop(0, maxc, stage, 0, unroll=False)

    def body(s, carry):
        xt = P[s]
        h_new, c_new = _lstm_update_cat(xt, hf[...], cf[...],
                                        Wf_ref, bf_ref, H)
        m = s < cnts
        hf[...] = jnp.where(m, h_new, hf[...])
        cf[...] = jnp.where(m, c_new, cf[...])
        tb = maxc - 1 - s
        xtb = P[tb]
        h_new, c_new = _lstm_update_cat(xtb, hb[...], cb[...],
                                        Wb_ref, bb_ref, H)
        mb = tb < cnts
        hb[...] = jnp.where(mb, h_new, hb[...])
        cb[...] = jnp.where(mb, c_new, cb[...])
        return carry

    jax.lax.fori_loop(0, maxc, body, 0, unroll=False)

    logits = (
        jnp.dot(hf[...], fcW_ref[:H, :], preferred_element_type=jnp.float32)
        + jnp.dot(hb[...], fcW_ref[H:, :], preferred_element_type=jnp.float32)
        + fcb_ref[0, 0]
    )
    out_ref[...] = _sigmoid(logits)


@jax.jit
def kernel(x, sWihf, sWhhf, sbf, sWihb, sWhhb, sbb, dWihf, dWhhf, dbf,
           dWihb, dWhhb, dbb, fcW, fcb, recover_idx, num_sent_per_document,
           sent_lengths):
    B = num_sent_per_document.shape[0]
    lens2d = sent_lengths.reshape(NS, 1)

    x_tm = jnp.transpose(x, (1, 0, 2))
    sent_emb_sorted = pl.pallas_call(
        _sent_kernel,
        grid=(NT,),
        in_specs=[
            pl.BlockSpec((NS, 1), lambda tt: (0, 0)),
            pl.BlockSpec((TSUB, NS, E), lambda tt: (tt, 0, 0)),
            pl.BlockSpec((TSUB, NS, E), lambda tt: (NT - 1 - tt, 0, 0)),
            pl.BlockSpec((E, 4 * H), lambda tt: (0, 0)),
            pl.BlockSpec((H, 4 * H), lambda tt: (0, 0)),
            pl.BlockSpec((1, 4 * H), lambda tt: (0, 0)),
            pl.BlockSpec((E, 4 * H), lambda tt: (0, 0)),
            pl.BlockSpec((H, 4 * H), lambda tt: (0, 0)),
            pl.BlockSpec((1, 4 * H), lambda tt: (0, 0)),
        ],
        out_specs=pl.BlockSpec((NS, 2 * H), lambda tt: (0, 0)),
        out_shape=jax.ShapeDtypeStruct((NS, 2 * H), jnp.float32),
        scratch_shapes=[pltpu.VMEM((NS, H), jnp.float32)] * 4,
    )(lens2d, x_tm, x_tm,
      sWihf, sWhhf, sbf.reshape(1, -1),
      sWihb, sWhhb, sbb.reshape(1, -1))

    counts = num_sent_per_document.astype(jnp.int32)
    offsets = jnp.concatenate(
        [jnp.zeros((1,), jnp.int32), jnp.cumsum(counts)[:-1]])
    maxc = jnp.max(counts).reshape(1)

    def catW(Wih, Whh):
        return jnp.concatenate([Wih, Whh]).astype(jnp.bfloat16)

    out2d = pl.pallas_call(
        _doc_kernel,
        in_specs=[
            pl.BlockSpec(memory_space=pltpu.SMEM),
            pl.BlockSpec(memory_space=pltpu.SMEM),
            pl.BlockSpec(memory_space=pltpu.SMEM),
            pl.BlockSpec((B, 1), lambda: (0, 0)),
            pl.BlockSpec((NS, 2 * H), lambda: (0, 0)),
            pl.BlockSpec((3 * H, 4 * H), lambda: (0, 0)),
            pl.BlockSpec((1, 4 * H), lambda: (0, 0)),
            pl.BlockSpec((3 * H, 4 * H), lambda: (0, 0)),
            pl.BlockSpec((1, 4 * H), lambda: (0, 0)),
            pl.BlockSpec((2 * H, 1), lambda: (0, 0)),
            pl.BlockSpec((1, 1), lambda: (0, 0)),
        ],
        out_specs=pl.BlockSpec((B, 1), lambda: (0, 0)),
        out_shape=jax.ShapeDtypeStruct((B, 1), jnp.float32),
        scratch_shapes=[pltpu.VMEM((NS, B, 2 * H), jnp.bfloat16)]
        + [pltpu.VMEM((B, H), jnp.float32)] * 4,
    )(recover_idx.astype(jnp.int32), offsets, maxc, counts.reshape(B, 1),
      sent_emb_sorted,
      catW(dWihf, dWhhf), dbf.reshape(1, -1),
      catW(dWihb, dWhhb), dbb.reshape(1, -1), fcW, fcb.reshape(1, 1))

    return out2d.reshape(-1)
